# Initial kernel scaffold; baseline (speedup 1.0000x reference)
import jax
import jax.numpy as jnp
from jax.experimental import pallas as pl


def kernel(x, x_node_feats, edge_attr, edge_index, batch, params):
    raise NotImplementedError("write your pallas kernel here")


# speedup vs baseline: 1.0184x; 1.0184x over previous
import functools

import jax
import jax.numpy as jnp
from jax.experimental import pallas as pl
from jax.experimental.pallas import tpu as pltpu

N = 50000
E = 800000
B = 64
IN = 119
CHEM = 11
H = 64
ED = 4
LAT = 64
NL = 3

ROW_BLK = 2000


def _silu(x):
    return x * jax.nn.sigmoid(x)


def _node_emb_body(x_ref, xn_ref, wx_ref, wn_ref, b_ref, o_ref):
    acc = jnp.dot(x_ref[...], wx_ref[...], preferred_element_type=jnp.float32)
    acc += jnp.dot(xn_ref[...], wn_ref[...], preferred_element_type=jnp.float32)
    acc += b_ref[...]
    o_ref[...] = acc * jax.nn.sigmoid(acc)


def _node_emb(x, xn, w, b):
    wx = w[:, :IN].T
    wn = w[:, IN:].T
    return pl.pallas_call(
        _node_emb_body,
        grid=(N // ROW_BLK,),
        in_specs=[
            pl.BlockSpec((ROW_BLK, IN), lambda i: (i, 0)),
            pl.BlockSpec((ROW_BLK, CHEM), lambda i: (i, 0)),
            pl.BlockSpec((IN, H), lambda i: (0, 0)),
            pl.BlockSpec((CHEM, H), lambda i: (0, 0)),
            pl.BlockSpec((1, H), lambda i: (0, 0)),
        ],
        out_specs=pl.BlockSpec((ROW_BLK, H), lambda i: (i, 0)),
        out_shape=jax.ShapeDtypeStruct((N, H), jnp.float32),
    )(x, xn, wx, wn, b.reshape(1, H))


def _lin(x, w, b):
    return x @ w.T + b


def kernel(x, x_node_feats, edge_attr, edge_index, batch, params):
    P = params
    h = _node_emb(x, x_node_feats, P["node_emb_w"], P["node_emb_b"])
    e = _lin(_silu(_lin(edge_attr, P["ee1_w"], P["ee1_b"])), P["ee2_w"], P["ee2_b"])
    src = edge_index[0]
    dst = edge_index[1]
    for l in range(NL):
        m = jax.nn.relu(h[src] + e)
        agg = jax.ops.segment_sum(m, dst, num_segments=N)
        hh = (1.0 + P["conv_eps"][l]) * h + agg
        hh = _lin(_silu(_lin(hh, P["conv_w1"][l], P["conv_b1"][l])), P["conv_w2"][l], P["conv_b2"][l])
        mu = jnp.mean(hh, axis=0)
        var = jnp.mean((hh - mu) ** 2, axis=0)
        hh = (hh - mu) / jnp.sqrt(var + 1e-5) * P["bn_g"][l] + P["bn_b"][l]
        h = _silu(hh)
    q_star = jnp.zeros((B, 2 * H), jnp.float32)
    hs = jnp.zeros((B, H), jnp.float32)
    cs = jnp.zeros((B, H), jnp.float32)
    for _ in range(3):
        z = q_star @ P["lstm_wih"].T + hs @ P["lstm_whh"].T + P["lstm_bih"] + P["lstm_bhh"]
        zi, zf, zg, zo = jnp.split(z, 4, axis=-1)
        cs = jax.nn.sigmoid(zf) * cs + jax.nn.sigmoid(zi) * jnp.tanh(zg)
        hs = jax.nn.sigmoid(zo) * jnp.tanh(cs)
        q = hs
        energy = jnp.sum(h * q[batch], axis=-1)
        emax = jax.ops.segment_max(energy, batch, num_segments=B)
        a = jnp.exp(energy - emax[batch])
        den = jax.ops.segment_sum(a, batch, num_segments=B)
        a = a / den[batch]
        r = jax.ops.segment_sum(a[:, None] * h, batch, num_segments=B)
        q_star = jnp.concatenate([q, r], axis=-1)
    latent = _lin(_silu(_lin(q_star, P["enc1_w"], P["enc1_b"])), P["enc2_w"], P["enc2_b"])
    gn = latent[batch]
    node_recon = _lin(_silu(_lin(gn, P["nd1_w"], P["nd1_b"])), P["nd2_w"], P["nd2_b"])
    edge_in = jnp.concatenate([gn[src], gn[dst]], axis=1)
    t = _silu(_lin(edge_in, P["ed1_w"], P["ed1_b"]))
    t = _silu(_lin(t, P["ed2_w"], P["ed2_b"]))
    er = _lin(t, P["ed3_w"], P["ed3_b"])
    edge_recon = jnp.concatenate([jax.nn.softplus(er[:, :1]), er[:, 1:]], axis=1)
    edge_logits = _lin(_silu(_lin(edge_in, P["ep1_w"], P["ep1_b"])), P["ep2_w"], P["ep2_b"])
    return (latent, node_recon, edge_logits, edge_recon)


# speedup vs baseline: 2.8053x; 2.7546x over previous
import functools

import jax
import jax.numpy as jnp
from jax import lax
from jax.experimental import pallas as pl
from jax.experimental.pallas import tpu as pltpu
from jax.experimental.pallas import tpu_sc as plsc

N = 50000
E = 800000
B = 64
IN = 119
CHEM = 11
H = 64
ED = 4
LAT = 64
NL = 3
HH = 32

NC = 2
NS = 16
EPS_BN = 1e-5

_f32 = jnp.float32


def _silu(x):
    return x * jax.nn.sigmoid(x)


_NODE_BLK = 2000


def _node_emb_body(x_ref, xn_ref, wx_ref, wn_ref, b_ref, o_ref):
    acc = jnp.dot(x_ref[...], wx_ref[...], preferred_element_type=_f32, precision=lax.Precision.HIGHEST)
    acc += jnp.dot(xn_ref[...], wn_ref[...], preferred_element_type=_f32, precision=lax.Precision.HIGHEST)
    acc += b_ref[...]
    hv = acc * jax.nn.sigmoid(acc)
    o_ref[0] = hv[:, :HH]
    o_ref[1] = hv[:, HH:]


def _node_emb(x, xn, w, b):
    return pl.pallas_call(
        _node_emb_body,
        grid=(N // _NODE_BLK,),
        in_specs=[
            pl.BlockSpec((_NODE_BLK, IN), lambda i: (i, 0)),
            pl.BlockSpec((_NODE_BLK, CHEM), lambda i: (i, 0)),
            pl.BlockSpec((IN, H), lambda i: (0, 0)),
            pl.BlockSpec((CHEM, H), lambda i: (0, 0)),
            pl.BlockSpec((1, H), lambda i: (0, 0)),
        ],
        out_specs=pl.BlockSpec((2, _NODE_BLK, HH), lambda i: (0, i, 0)),
        out_shape=jax.ShapeDtypeStruct((2, N, HH), _f32),
    )(x, xn, w[:, :IN].T, w[:, IN:].T, b.reshape(1, H))


_EDGE_BLK = 8000


def _edge_emb_body(a_ref, w1_ref, b1_ref, w2_ref, b2_ref, o_ref):
    a = a_ref[...]
    z = b1_ref[...]
    z = z + jnp.dot(a, w1_ref[...], preferred_element_type=_f32, precision=lax.Precision.HIGHEST)
    z = z * jax.nn.sigmoid(z)
    e = jnp.dot(z, w2_ref[...], preferred_element_type=_f32, precision=lax.Precision.HIGHEST) + b2_ref[...]
    o_ref[0] = e[:, :HH]
    o_ref[1] = e[:, HH:]


def _edge_emb(edge_attr, w1, b1, w2, b2):
    return pl.pallas_call(
        _edge_emb_body,
        grid=(E // _EDGE_BLK,),
        in_specs=[
            pl.BlockSpec((_EDGE_BLK, ED), lambda i: (i, 0)),
            pl.BlockSpec((ED, H), lambda i: (0, 0)),
            pl.BlockSpec((1, H), lambda i: (0, 0)),
            pl.BlockSpec((H, H), lambda i: (0, 0)),
            pl.BlockSpec((1, H), lambda i: (0, 0)),
        ],
        out_specs=pl.BlockSpec((2, _EDGE_BLK, HH), lambda i: (0, i, 0)),
        out_shape=jax.ShapeDtypeStruct((2, E, HH), _f32),
    )(edge_attr, w1.T, b1.reshape(1, H), w2.T, b2.reshape(1, H))


_C = 128
_EPW = E // NS
_NCHUNK = 390
_CT = _EPW - _NCHUNK * _C
_SROWS = 3128
_NPAD = NS * _SROWS
_LROWS = N - 15 * _SROWS


def _gine_body(hcat, ecat, src, dst, out, agg_sh,
               sidx0, sidx1, didx0, didx1, gidx0, gidx1,
               hbuf0, hbuf1, ebuf0, ebuf1,
               tsidx, tdidx, thbuf, tebuf,
               hsem0, hsem1, esem0, esem1):
    core = lax.axis_index("c")
    sub = lax.axis_index("s")
    ebase = core * E + sub * _EPW
    ibase = sub * _EPW
    hoff = core * N

    sidx = (sidx0, sidx1)
    didx = (didx0, didx1)
    gidx = (gidx0, gidx1)
    hbuf = (hbuf0, hbuf1)
    ebuf = (ebuf0, ebuf1)
    hsem = (hsem0, hsem1)
    esem = (esem0, esem1)

    zeros16 = jnp.zeros((16,), _f32)

    @pl.loop(0, _C)
    def _(r):
        hbuf0[r, pl.ds(0, 16)] = zeros16
        hbuf0[r, pl.ds(16, 16)] = zeros16

    @pl.loop(0, 24)
    def _(k):
        pltpu.sync_copy(hbuf0, agg_sh.at[pl.ds(sub * _SROWS + k * _C, _C)])

    pltpu.sync_copy(hbuf0.at[pl.ds(0, _SROWS - 24 * _C)],
                    agg_sh.at[pl.ds(sub * _SROWS + 24 * _C, _SROWS - 24 * _C)])

    plsc.subcore_barrier()

    def load(i, b):
        pltpu.sync_copy(src.at[pl.ds(ibase + i * _C, _C)], sidx[b])
        pltpu.sync_copy(dst.at[pl.ds(ibase + i * _C, _C)], didx[b])

        @pl.loop(0, _C // 16)
        def _(j):
            gidx[b][pl.ds(j * 16, 16)] = sidx[b][pl.ds(j * 16, 16)] + hoff

        pltpu.async_copy(hcat.at[gidx[b]], hbuf[b], hsem[b])
        pltpu.async_copy(ecat.at[pl.ds(ebase + i * _C, _C)], ebuf[b], esem[b])

    def finish(b):
        pltpu.make_async_copy(hcat.at[gidx[b]], hbuf[b], hsem[b]).wait()
        pltpu.make_async_copy(ecat.at[pl.ds(ebase, _C)], ebuf[b], esem[b]).wait()

        @pl.loop(0, _C)
        def _(r):
            v0 = hbuf[b][r, pl.ds(0, 16)] + ebuf[b][r, pl.ds(0, 16)]
            hbuf[b][r, pl.ds(0, 16)] = jnp.maximum(v0, 0.0)
            v1 = hbuf[b][r, pl.ds(16, 16)] + ebuf[b][r, pl.ds(16, 16)]
            hbuf[b][r, pl.ds(16, 16)] = jnp.maximum(v1, 0.0)

        pltpu.sync_copy(hbuf[b], agg_sh.at[didx[b]], add=True)

    load(0, 0)

    @pl.loop(0, _NCHUNK, step=2)
    def _(c):
        load(c + 1, 1)
        finish(0)

        @pl.when(c + 2 < _NCHUNK)
        def _():
            load(c + 2, 0)

        finish(1)

    tbase = ibase + _NCHUNK * _C
    pltpu.sync_copy(src.at[pl.ds(tbase, _CT)], tsidx)
    pltpu.sync_copy(dst.at[pl.ds(tbase, _CT)], tdidx)

    @pl.loop(0, _CT // 16)
    def _(j):
        tsidx[pl.ds(j * 16, 16)] = tsidx[pl.ds(j * 16, 16)] + hoff

    pltpu.async_copy(hcat.at[tsidx], thbuf, hsem0).wait()
    pltpu.async_copy(ecat.at[pl.ds(core * E + tbase, _CT)], tebuf, esem0).wait()

    @pl.loop(0, _CT)
    def _(r):
        v0 = thbuf[r, pl.ds(0, 16)] + tebuf[r, pl.ds(0, 16)]
        thbuf[r, pl.ds(0, 16)] = jnp.maximum(v0, 0.0)
        v1 = thbuf[r, pl.ds(16, 16)] + tebuf[r, pl.ds(16, 16)]
        thbuf[r, pl.ds(16, 16)] = jnp.maximum(v1, 0.0)

    pltpu.sync_copy(thbuf, agg_sh.at[tdidx], add=True)

    plsc.subcore_barrier()

    @pl.when(sub < NS - 1)
    def _():
        pltpu.sync_copy(agg_sh.at[pl.ds(sub * _SROWS, _SROWS)],
                        out.at[pl.ds(core * N + sub * _SROWS, _SROWS)])

    @pl.when(sub == NS - 1)
    def _():
        pltpu.sync_copy(agg_sh.at[pl.ds(sub * _SROWS, _LROWS)],
                        out.at[pl.ds(core * N + sub * _SROWS, _LROWS)])


@functools.cache
def _get_gine_kernel():
    return pl.kernel(
        _gine_body,
    out_type=jax.ShapeDtypeStruct((2 * N, HH), _f32),
    mesh=plsc.VectorSubcoreMesh(core_axis_name="c", subcore_axis_name="s",
                                num_cores=NC, num_subcores=NS),
    scratch_types=[
        pltpu.VMEM_SHARED((_NPAD, HH), _f32),
        pltpu.VMEM((_C,), jnp.int32), pltpu.VMEM((_C,), jnp.int32),
        pltpu.VMEM((_C,), jnp.int32), pltpu.VMEM((_C,), jnp.int32),
        pltpu.VMEM((_C,), jnp.int32), pltpu.VMEM((_C,), jnp.int32),
        pltpu.VMEM((_C, HH), _f32), pltpu.VMEM((_C, HH), _f32),
        pltpu.VMEM((_C, HH), _f32), pltpu.VMEM((_C, HH), _f32),
        pltpu.VMEM((_CT,), jnp.int32), pltpu.VMEM((_CT,), jnp.int32),
        pltpu.VMEM((_CT, HH), _f32), pltpu.VMEM((_CT, HH), _f32),
        pltpu.SemaphoreType.DMA, pltpu.SemaphoreType.DMA,
        pltpu.SemaphoreType.DMA, pltpu.SemaphoreType.DMA,
    ],
        compiler_params=pltpu.CompilerParams(use_tc_tiling_on_sc=False),
    )


_CB = 2000
_CNB = N // _CB


def _conv_a_body(a0_ref, a1_ref, h0_ref, h1_ref, eps_ref,
                 w1a_ref, w1b_ref, b1_ref, w2_ref, b2_ref,
                 z_ref, st_ref, acc_ref):
    i = pl.program_id(0)
    s = 1.0 + eps_ref[0, 0]
    u0 = s * h0_ref[...] + a0_ref[...]
    u1 = s * h1_ref[...] + a1_ref[...]
    z = b1_ref[...]
    z = z + jnp.dot(u0, w1a_ref[...], preferred_element_type=_f32, precision=lax.Precision.HIGHEST)
    z = z + jnp.dot(u1, w1b_ref[...], preferred_element_type=_f32, precision=lax.Precision.HIGHEST)
    z = z * jax.nn.sigmoid(z)
    z = jnp.dot(z, w2_ref[...], preferred_element_type=_f32, precision=lax.Precision.HIGHEST) + b2_ref[...]
    z_ref[...] = z

    @pl.when(i == 0)
    def _():
        acc_ref[0:2, :] = jnp.zeros((2, H), _f32)
        acc_ref[2:3, :] = jnp.mean(z, axis=0, keepdims=True)

    zc = z - acc_ref[2:3, :]
    acc_ref[0:1, :] += jnp.sum(zc, axis=0, keepdims=True)
    acc_ref[1:2, :] += jnp.sum(zc * zc, axis=0, keepdims=True)

    @pl.when(i == _CNB - 1)
    def _():
        st_ref[...] = acc_ref[...]


def _conv_b_body(z_ref, st_ref, g_ref, bb_ref, o_ref):
    s0 = st_ref[0:1, :] / N
    mu = st_ref[2:3, :] + s0
    var = st_ref[1:2, :] / N - s0 * s0
    hv = (z_ref[...] - mu) * jax.lax.rsqrt(var + EPS_BN) * g_ref[...] + bb_ref[...]
    hv = hv * jax.nn.sigmoid(hv)
    o_ref[0] = hv[:, :HH]
    o_ref[1] = hv[:, HH:]


def _conv_mlp(aggcat, hcat, eps_l, w1, b1, w2, b2, g, bb):
    z, st = pl.pallas_call(
        _conv_a_body,
        grid=(_CNB,),
        in_specs=[
            pl.BlockSpec((_CB, HH), lambda i: (i, 0)),
            pl.BlockSpec((_CB, HH), lambda i: (N // _CB + i, 0)),
            pl.BlockSpec((_CB, HH), lambda i: (i, 0)),
            pl.BlockSpec((_CB, HH), lambda i: (N // _CB + i, 0)),
            pl.BlockSpec((1, 1), lambda i: (0, 0)),
            pl.BlockSpec((HH, H), lambda i: (0, 0)),
            pl.BlockSpec((HH, H), lambda i: (0, 0)),
            pl.BlockSpec((1, H), lambda i: (0, 0)),
            pl.BlockSpec((H, H), lambda i: (0, 0)),
            pl.BlockSpec((1, H), lambda i: (0, 0)),
        ],
        out_specs=[
            pl.BlockSpec((_CB, H), lambda i: (i, 0)),
            pl.BlockSpec((3, H), lambda i: (0, 0)),
        ],
        out_shape=[
            jax.ShapeDtypeStruct((N, H), _f32),
            jax.ShapeDtypeStruct((3, H), _f32),
        ],
        scratch_shapes=[pltpu.VMEM((3, H), _f32)],
    )(aggcat, aggcat, hcat, hcat, eps_l,
      w1[:, :HH].T, w1[:, HH:].T, b1.reshape(1, H), w2.T, b2.reshape(1, H))
    return pl.pallas_call(
        _conv_b_body,
        grid=(_CNB,),
        in_specs=[
            pl.BlockSpec((_CB, H), lambda i: (i, 0)),
            pl.BlockSpec((3, H), lambda i: (0, 0)),
            pl.BlockSpec((1, H), lambda i: (0, 0)),
            pl.BlockSpec((1, H), lambda i: (0, 0)),
        ],
        out_specs=pl.BlockSpec((2, _CB, HH), lambda i: (0, i, 0)),
        out_shape=jax.ShapeDtypeStruct((2, N, HH), _f32),
    )(z, st, g.reshape(1, H), bb.reshape(1, H))


_S2S_CHUNK = 2000


_S2S_NCH = N // _S2S_CHUNK


def _set2set_body(h0_ref, h1_ref, bat_ref, wih_ref, whh_ref, bias_ref,
                  q_ref, emax_s, den_s, rnum_s, q_s, qstar_s, hs_s, cs_s):
    gid = pl.program_id(0)
    phase = (gid // _S2S_NCH) % 2
    chunk = gid % _S2S_NCH
    iota_b = lax.broadcasted_iota(jnp.int32, (1, B), 1)

    @pl.when(gid == 0)
    def _():
        qstar_s[...] = jnp.zeros((B, 2 * H), _f32)
        hs_s[...] = jnp.zeros((B, H), _f32)
        cs_s[...] = jnp.zeros((B, H), _f32)

    @pl.when(jnp.logical_and(phase == 0, chunk == 0))
    def _():
        z = (jnp.dot(qstar_s[...], wih_ref[...], preferred_element_type=_f32, precision=lax.Precision.HIGHEST)
             + jnp.dot(hs_s[...], whh_ref[...], preferred_element_type=_f32, precision=lax.Precision.HIGHEST)
             + bias_ref[...])
        zi, zf, zg, zo = (z[:, :H], z[:, H:2 * H], z[:, 2 * H:3 * H], z[:, 3 * H:])
        cs = jax.nn.sigmoid(zf) * cs_s[...] + jax.nn.sigmoid(zi) * jnp.tanh(zg)
        hs = jax.nn.sigmoid(zo) * jnp.tanh(cs)
        cs_s[...] = cs
        hs_s[...] = hs
        q_s[...] = hs
        emax_s[...] = jnp.full((1, B), -1e30, _f32)

    oh = (bat_ref[...] == iota_b).astype(_f32)
    hch = jnp.concatenate([h0_ref[...], h1_ref[...]], axis=1)
    qb = jnp.dot(oh, q_s[...], preferred_element_type=_f32, precision=lax.Precision.HIGHEST)
    en = jnp.sum(hch * qb, axis=1, keepdims=True)

    @pl.when(phase == 0)
    def _():
        msk = jnp.where(oh > 0.0, en, -1e30)
        emax_s[...] = jnp.maximum(emax_s[...], jnp.max(msk, axis=0, keepdims=True))

    @pl.when(phase == 1)
    def _():
        @pl.when(chunk == 0)
        def _():
            den_s[...] = jnp.zeros((1, B), _f32)
            rnum_s[...] = jnp.zeros((B, H), _f32)

        eb = jnp.sum(oh * emax_s[...], axis=1, keepdims=True)
        a = jnp.exp(en - eb)
        den_s[...] += jnp.sum(a * oh, axis=0, keepdims=True)
        rnum_s[...] += lax.dot_general(oh, a * hch, (((0,), (0,)), ((), ())),
                                       preferred_element_type=_f32, precision=lax.Precision.HIGHEST)

        @pl.when(chunk == _S2S_NCH - 1)
        def _():
            r = rnum_s[...] / jnp.maximum(den_s[...], 1e-30).T
            qs = jnp.concatenate([q_s[...], r], axis=1)
            qstar_s[...] = qs
            q_ref[...] = qs


def _set2set(hcat, batch2d, wih, whh, bias):
    def h_idx(g, off):
        return (off + (g % _S2S_NCH), 0)

    return pl.pallas_call(
        _set2set_body,
        grid=(3 * 2 * _S2S_NCH,),
        in_specs=[
            pl.BlockSpec((_S2S_CHUNK, HH), lambda g: h_idx(g, 0)),
            pl.BlockSpec((_S2S_CHUNK, HH), lambda g: h_idx(g, N // _S2S_CHUNK)),
            pl.BlockSpec((_S2S_CHUNK, 1), lambda g: (g % _S2S_NCH, 0)),
            pl.BlockSpec((2 * H, 4 * H), lambda g: (0, 0)),
            pl.BlockSpec((H, 4 * H), lambda g: (0, 0)),
            pl.BlockSpec((1, 4 * H), lambda g: (0, 0)),
        ],
        out_specs=pl.BlockSpec((B, 2 * H), lambda g: (0, 0)),
        out_shape=jax.ShapeDtypeStruct((B, 2 * H), _f32),
        scratch_shapes=[
            pltpu.VMEM((1, B), _f32), pltpu.VMEM((1, B), _f32),
            pltpu.VMEM((B, H), _f32), pltpu.VMEM((B, H), _f32),
            pltpu.VMEM((B, 2 * H), _f32), pltpu.VMEM((B, H), _f32),
            pltpu.VMEM((B, H), _f32),
        ],
    )(hcat, hcat, batch2d, wih.T, whh.T, bias.reshape(1, 4 * H))


def _tables_body(q_ref, e1w_ref, e1b_ref, e2w_ref, e2b_ref,
                 n1w_ref, n1b_ref, n2w_ref, n2b_ref,
                 d1a_ref, d1b_ref, d1bias_ref, d2w_ref, d2b_ref,
                 d3w_ref, d3b_ref,
                 p1a_ref, p1b_ref, p1bias_ref, p2w_ref, p2b_ref,
                 lat_ref, ntab_ref, etab_ref):
    t = jnp.dot(q_ref[...], e1w_ref[...], preferred_element_type=_f32, precision=lax.Precision.HIGHEST) + e1b_ref[...]
    t = t * jax.nn.sigmoid(t)
    lat = jnp.dot(t, e2w_ref[...], preferred_element_type=_f32, precision=lax.Precision.HIGHEST) + e2b_ref[...]
    lat_ref[...] = lat

    t = jnp.dot(lat, n1w_ref[...], preferred_element_type=_f32, precision=lax.Precision.HIGHEST) + n1b_ref[...]
    t = t * jax.nn.sigmoid(t)
    ntab_ref[...] = jnp.dot(t, n2w_ref[...], preferred_element_type=_f32, precision=lax.Precision.HIGHEST) + n2b_ref[...]

    la = jnp.broadcast_to(lat[:, None, :], (B, B, LAT)).reshape(B * B, LAT)
    lb = jnp.broadcast_to(lat[None, :, :], (B, B, LAT)).reshape(B * B, LAT)
    t = (jnp.dot(la, d1a_ref[...], preferred_element_type=_f32, precision=lax.Precision.HIGHEST)
         + jnp.dot(lb, d1b_ref[...], preferred_element_type=_f32, precision=lax.Precision.HIGHEST) + d1bias_ref[...])
    t = t * jax.nn.sigmoid(t)
    t = jnp.dot(t, d2w_ref[...], preferred_element_type=_f32, precision=lax.Precision.HIGHEST) + d2b_ref[...]
    t = t * jax.nn.sigmoid(t)
    er = jnp.dot(t, d3w_ref[...], preferred_element_type=_f32, precision=lax.Precision.HIGHEST) + d3b_ref[...]

    t = (jnp.dot(la, p1a_ref[...], preferred_element_type=_f32, precision=lax.Precision.HIGHEST)
         + jnp.dot(lb, p1b_ref[...], preferred_element_type=_f32, precision=lax.Precision.HIGHEST) + p1bias_ref[...])
    t = t * jax.nn.sigmoid(t)
    lg = jnp.dot(t, p2w_ref[...], preferred_element_type=_f32, precision=lax.Precision.HIGHEST) + p2b_ref[...]

    er0 = er[:, 0:1]
    sp = jnp.maximum(er0, 0.0) + jnp.log(1.0 + jnp.exp(-jnp.abs(er0)))
    pad = jnp.zeros((B * B, 3), _f32)
    etab_ref[...] = jnp.concatenate([sp, er[:, 1:], lg, pad], axis=1)


def _tables(q_star, P):
    return pl.pallas_call(
        _tables_body,
        grid=(1,),
        in_specs=[pl.BlockSpec(idx.shape, lambda i: tuple(0 for _ in idx.shape))
                  for idx in (
                      jnp.zeros((B, 2 * H)), jnp.zeros((2 * H, H)), jnp.zeros((1, H)),
                      jnp.zeros((H, LAT)), jnp.zeros((1, LAT)),
                      jnp.zeros((LAT, H // 2)), jnp.zeros((1, H // 2)),
                      jnp.zeros((H // 2, IN)), jnp.zeros((1, IN)),
                      jnp.zeros((LAT, H)), jnp.zeros((LAT, H)), jnp.zeros((1, H)),
                      jnp.zeros((H, H)), jnp.zeros((1, H)),
                      jnp.zeros((H, ED)), jnp.zeros((1, ED)),
                      jnp.zeros((LAT, H)), jnp.zeros((LAT, H)), jnp.zeros((1, H)),
                      jnp.zeros((H, 1)), jnp.zeros((1, 1)),
                  )],
        out_specs=[
            pl.BlockSpec((B, LAT), lambda i: (0, 0)),
            pl.BlockSpec((B, IN), lambda i: (0, 0)),
            pl.BlockSpec((B * B, 8), lambda i: (0, 0)),
        ],
        out_shape=[
            jax.ShapeDtypeStruct((B, LAT), _f32),
            jax.ShapeDtypeStruct((B, IN), _f32),
            jax.ShapeDtypeStruct((B * B, 8), _f32),
        ],
    )(q_star,
      P["enc1_w"].T, P["enc1_b"].reshape(1, H),
      P["enc2_w"].T, P["enc2_b"].reshape(1, LAT),
      P["nd1_w"].T, P["nd1_b"].reshape(1, H // 2),
      P["nd2_w"].T, P["nd2_b"].reshape(1, IN),
      P["ed1_w"][:, :LAT].T, P["ed1_w"][:, LAT:].T, P["ed1_b"].reshape(1, H),
      P["ed2_w"].T, P["ed2_b"].reshape(1, H),
      P["ed3_w"].T, P["ed3_b"].reshape(1, ED),
      P["ep1_w"][:, :LAT].T, P["ep1_w"][:, LAT:].T, P["ep1_b"].reshape(1, H),
      P["ep2_w"].T, P["ep2_b"].reshape(1, 1))


def _node_recon_body(bat_ref, tab_ref, o_ref):
    iota_b = lax.broadcasted_iota(jnp.int32, (1, B), 1)
    oh = (bat_ref[...] == iota_b).astype(_f32)
    o_ref[...] = jnp.dot(oh, tab_ref[...], preferred_element_type=_f32, precision=lax.Precision.HIGHEST)


def _node_recon(batch2d, ntab):
    return pl.pallas_call(
        _node_recon_body,
        grid=(N // _NODE_BLK,),
        in_specs=[
            pl.BlockSpec((_NODE_BLK, 1), lambda i: (i, 0)),
            pl.BlockSpec((B, IN), lambda i: (0, 0)),
        ],
        out_specs=pl.BlockSpec((_NODE_BLK, IN), lambda i: (i, 0)),
        out_shape=jax.ShapeDtypeStruct((N, IN), _f32),
    )(batch2d, ntab)


_DK = 1000
_DEPW = E // (NC * NS)
_DNCH = _DEPW // _DK
_DG = (_DK + 15) // 16


def _edec_body(src, dst, batch, tab, er_out, lg_out,
               bat_v, tab_v, sbuf, dbuf, obuf, lbuf):
    core = lax.axis_index("c")
    sub = lax.axis_index("s")
    wid = sub * NC + core
    base = wid * _DEPW

    pltpu.sync_copy(batch, bat_v)
    pltpu.sync_copy(tab, tab_v)

    iota16 = lax.broadcasted_iota(jnp.int32, (16,), 0)

    @pl.loop(0, _DNCH)
    def _(ci):
        pltpu.sync_copy(src.at[pl.ds(base + ci * _DK, _DK)], sbuf)
        pltpu.sync_copy(dst.at[pl.ds(base + ci * _DK, _DK)], dbuf)

        @pl.loop(0, _DG)
        def _(g):
            idx = jnp.minimum(iota16 + g * 16, _DK - 1)
            sv = plsc.load_gather(sbuf, [idx])
            dv = plsc.load_gather(dbuf, [idx])
            gs = plsc.load_gather(bat_v, [sv])
            gd = plsc.load_gather(bat_v, [dv])
            pair8 = (gs * B + gd) * 8
            rows4 = idx * 4
            for j in range(4):
                v = plsc.load_gather(tab_v, [pair8 + j])
                plsc.store_scatter(obuf, [rows4 + j], v)
            v = plsc.load_gather(tab_v, [pair8 + 4])
            plsc.store_scatter(lbuf, [idx], v)

        pltpu.sync_copy(obuf, er_out.at[pl.ds((base + ci * _DK) * 4, _DK * 4)])
        pltpu.sync_copy(lbuf, lg_out.at[pl.ds(base + ci * _DK, _DK)])


@functools.cache
def _get_edec_kernel():
    return pl.kernel(
        _edec_body,
    out_type=[jax.ShapeDtypeStruct((E * 4,), _f32),
              jax.ShapeDtypeStruct((E,), _f32)],
    mesh=plsc.VectorSubcoreMesh(core_axis_name="c", subcore_axis_name="s",
                                num_cores=NC, num_subcores=NS),
    scratch_types=[
        pltpu.VMEM((N,), jnp.int32),
        pltpu.VMEM((B * B * 8,), _f32),
        pltpu.VMEM((_DK,), jnp.int32), pltpu.VMEM((_DK,), jnp.int32),
        pltpu.VMEM((_DK * 4,), _f32), pltpu.VMEM((_DK,), _f32),
    ],
        compiler_params=pltpu.CompilerParams(needs_layout_passes=False,
                                             use_tc_tiling_on_sc=False),
    )


def kernel(x, x_node_feats, edge_attr, edge_index, batch, params):
    P = params
    src = edge_index[0]
    dst = edge_index[1]
    batch2d = batch.reshape(N, 1)

    hcat = _node_emb(x, x_node_feats, P["node_emb_w"], P["node_emb_b"])
    ecat = _edge_emb(edge_attr, P["ee1_w"], P["ee1_b"], P["ee2_w"], P["ee2_b"])
    ecat2 = ecat.reshape(2 * E, HH)

    for l in range(NL):
        hflat = hcat.reshape(2 * N, HH)
        agg = _get_gine_kernel()(hflat, ecat2, src, dst)
        hcat = _conv_mlp(agg, hflat,
                         P["conv_eps"][l].reshape(1, 1),
                         P["conv_w1"][l], P["conv_b1"][l],
                         P["conv_w2"][l], P["conv_b2"][l],
                         P["bn_g"][l], P["bn_b"][l])

    q_star = _set2set(hcat.reshape(2 * N, HH), batch2d,
                      P["lstm_wih"], P["lstm_whh"], P["lstm_bih"] + P["lstm_bhh"])
    latent, ntab, etab = _tables(q_star, P)
    node_recon = _node_recon(batch2d, ntab)
    er, lg = _get_edec_kernel()(src, dst, batch, etab.reshape(B * B * 8))
    edge_logits = lg.reshape(E, 1)
    return (latent, node_recon, edge_logits, er.reshape(E, 4))
